# baseline (speedup 1.0000x reference)
import functools

import jax
import jax.numpy as jnp
from jax import lax
from jax.experimental import pallas as pl
from jax.experimental.pallas import tpu as pltpu
from jax.experimental.pallas import tpu_sc as plsc

VOCAB = 1000000
HIDDEN = 128
SEQ_LEN = 16384

_NC = 2
_NS = 16
_NW = _NC * _NS

_B_PER_W = SEQ_LEN // _NW
_CHUNK = 128
_NCHUNK = _B_PER_W // _CHUNK
_PAD = 8


def _make_gather():
    mesh = plsc.VectorSubcoreMesh(core_axis_name="c", subcore_axis_name="s")

    @functools.partial(
        pl.kernel,
        out_type=jax.ShapeDtypeStruct((SEQ_LEN + _PAD, HIDDEN), jnp.float32),
        mesh=mesh,
        scratch_types=[
            pltpu.VMEM((_NCHUNK, _CHUNK), jnp.int32),
            pltpu.VMEM((_B_PER_W, HIDDEN), jnp.float32),
            pltpu.VMEM((HIDDEN,), jnp.float32),
            pltpu.SemaphoreType.DMA,
        ],
    )
    def gather_kernel(idx_hbm, table_hbm, out_hbm, idx_v, rows_v, zero_v, sem):
        wid = lax.axis_index("s") * _NC + lax.axis_index("c")
        base = wid * _B_PER_W
        pltpu.sync_copy(idx_hbm.at[wid], idx_v)
        gathers = [
            pltpu.async_copy(
                table_hbm.at[idx_v.at[j]],
                rows_v.at[pl.ds(j * _CHUNK, _CHUNK)],
                sem,
            )
            for j in range(_NCHUNK)
        ]
        for g in gathers:
            g.wait()
        @pl.when(wid == 0)
        def _():
            z = jnp.zeros((16,), jnp.float32)
            for i in range(HIDDEN // 16):
                zero_v[pl.ds(i * 16, 16)] = z
            pltpu.sync_copy(zero_v, out_hbm.at[SEQ_LEN])

        pltpu.sync_copy(rows_v, out_hbm.at[pl.ds(base, _B_PER_W)])

    return gather_kernel


_gather = _make_gather()


def kernel(word_inputs, hidden, embedding_weight):
    idx = word_inputs.astype(jnp.int32).reshape(_NW, _NCHUNK, _CHUNK)
    buf = _gather(idx, embedding_weight)
    return (
        buf[:SEQ_LEN].reshape(SEQ_LEN, 1, HIDDEN),
        buf[SEQ_LEN : SEQ_LEN + 1].reshape(1, 1, HIDDEN),
    )


# speedup vs baseline: 1.2418x; 1.2418x over previous
import functools

import jax
import jax.numpy as jnp
from jax import lax
from jax.experimental import pallas as pl
from jax.experimental.pallas import tpu as pltpu
from jax.experimental.pallas import tpu_sc as plsc

VOCAB = 1000000
HIDDEN = 128
SEQ_LEN = 16384

_NC = 2
_NS = 16
_NW = _NC * _NS

_B_PER_W = SEQ_LEN // _NW
_CHUNK = 128
_NCHUNK = _B_PER_W // _CHUNK


def _make_gather():
    mesh = plsc.VectorSubcoreMesh(core_axis_name="c", subcore_axis_name="s")

    @functools.partial(
        pl.kernel,
        out_type=(
            jax.ShapeDtypeStruct((_NW, _NCHUNK, _CHUNK, HIDDEN), jnp.float32),
            jax.ShapeDtypeStruct((HIDDEN,), jnp.float32),
        ),
        mesh=mesh,
        scratch_types=[
            pltpu.VMEM((_NCHUNK, _CHUNK), jnp.int32),
            pltpu.VMEM((_NCHUNK, _CHUNK, HIDDEN), jnp.float32),
            pltpu.VMEM((HIDDEN,), jnp.float32),
            pltpu.SemaphoreType.DMA,
        ],
    )
    def gather_kernel(idx_hbm, table_hbm, out_hbm, hid_hbm, idx_v, rows_v,
                      zero_v, sem):
        wid = lax.axis_index("s") * _NC + lax.axis_index("c")
        pltpu.sync_copy(idx_hbm.at[wid], idx_v)
        gathers = [
            pltpu.async_copy(table_hbm.at[idx_v.at[j]], rows_v.at[j], sem)
            for j in range(_NCHUNK)
        ]
        for g in gathers:
            g.wait()
        @pl.when(wid == 0)
        def _():
            z = jnp.zeros((16,), jnp.float32)
            for i in range(HIDDEN // 16):
                zero_v[pl.ds(i * 16, 16)] = z
            pltpu.sync_copy(zero_v, hid_hbm)

        pltpu.sync_copy(rows_v, out_hbm.at[wid])

    return gather_kernel


_gather = _make_gather()


def kernel(word_inputs, hidden, embedding_weight):
    idx = word_inputs.astype(jnp.int32).reshape(_NW, _NCHUNK, _CHUNK)
    embedded, hid = _gather(idx, embedding_weight)
    return (
        embedded.reshape(SEQ_LEN, 1, HIDDEN),
        hid.reshape(1, 1, HIDDEN),
    )
